# baseline (speedup 1.0000x reference)
import functools

import jax
import jax.numpy as jnp
from jax import lax
from jax.experimental import pallas as pl
from jax.experimental.pallas import tpu as pltpu
from jax.experimental.pallas import tpu_sc as plsc

_COMMIT_W = 0.25


def _vq_tc_body(z_ref, cb_ref, idx_ref, acc_ref):
    first = jnp.logical_and(pl.program_id(0) == 0, pl.program_id(1) == 0)

    @pl.when(first)
    def _():
        acc_ref[...] = jnp.zeros((1, 1), jnp.float32)

    zt = z_ref[0]
    cb = cb_ref[...]
    k = cb.shape[0]

    scores2 = lax.dot_general(cb, -2.0 * zt, (((1,), (0,)), ((), ())),
                              preferred_element_type=jnp.float32)
    cbsq = jnp.sum(cb * cb, axis=1, keepdims=True)
    dist = scores2 + cbsq

    m = jnp.min(dist, axis=0, keepdims=True)
    ind = (dist == m).astype(jnp.float32)

    kcol = lax.broadcasted_iota(jnp.int32, (k, 1), 0)
    khi = (kcol // 32).astype(jnp.float32)
    klo = (kcol % 32).astype(jnp.float32)
    kaug = jnp.concatenate([khi, klo], axis=1)
    qa = lax.dot_general(kaug, ind, (((0,), (0,)), ((), ())),
                         preferred_element_type=jnp.float32)
    idx = (qa[0:1, :] * 32.0 + qa[1:2, :]).astype(jnp.int32)

    acc_ref[...] += (jnp.sum(m) + jnp.sum(zt * zt)).reshape(1, 1)
    idx_ref[0] = idx


def _tc_indices(zc, codebook):
    b, c, n = zc.shape
    tile = 4096
    n_t = n // tile
    k = codebook.shape[0]
    idx_arr, acc = pl.pallas_call(
        _vq_tc_body,
        grid=(b, n_t),
        in_specs=[
            pl.BlockSpec((1, c, tile), lambda i, j: (i, 0, j)),
            pl.BlockSpec((k, c), lambda i, j: (0, 0)),
        ],
        out_specs=[
            pl.BlockSpec((1, 1, tile), lambda i, j: (i * (n // tile) + j, 0, 0)),
            pl.BlockSpec((1, 1), lambda i, j: (0, 0)),
        ],
        out_shape=[
            jax.ShapeDtypeStruct((b * n_t, 1, tile), jnp.int32),
            jax.ShapeDtypeStruct((1, 1), jnp.float32),
        ],
    )(zc, codebook)
    return idx_arr.reshape(b * n), acc


def _sc_gather_body(idx_hbm, cbt_hbm, out_hbm, idx_v, cbt_v, stage_v):
    wid = lax.axis_index("s") * 2 + lax.axis_index("c")
    ntok = 2048
    base = wid * ntok
    bi = base // 16384
    off = base % 16384
    pltpu.sync_copy(idx_hbm.at[pl.ds(base, ntok)], idx_v)
    pltpu.sync_copy(cbt_hbm, cbt_v)

    chunk = 256

    def chunk_body(ch, carry):
        def grp(g, carry2):
            iv = idx_v[pl.ds(ch * chunk + g * 16, 16)]
            for cc in range(32):
                cvec = jnp.full((16,), cc, jnp.int32)
                stage_v[cc, pl.ds(g * 16, 16)] = plsc.load_gather(
                    cbt_v, [cvec, iv])
            return carry2
        lax.fori_loop(0, chunk // 16, grp, 0)
        dst = pl.multiple_of(off + ch * chunk, chunk)
        pltpu.sync_copy(stage_v, out_hbm.at[bi, :, pl.ds(dst, chunk)])
        return carry
    lax.fori_loop(0, ntok // chunk, chunk_body, 0)


def _sc_gather(idx_flat, codebook_t, b, c, n):
    mesh = plsc.VectorSubcoreMesh(core_axis_name="c", subcore_axis_name="s")
    f = functools.partial(
        pl.kernel,
        out_type=jax.ShapeDtypeStruct((b, c, n), jnp.float32),
        compiler_params=pltpu.CompilerParams(needs_layout_passes=False),
        mesh=mesh,
        scratch_types=[
            pltpu.VMEM((2048,), jnp.int32),
            pltpu.VMEM((c, codebook_t.shape[1]), jnp.float32),
            pltpu.VMEM((c, 256), jnp.float32),
        ],
    )(_sc_gather_body)
    return f(idx_flat, codebook_t)


@jax.jit
def kernel(z, codebook):
    b, c, f, h, w = z.shape
    n = f * h * w
    zc = z.reshape(b, c, n)

    idx_flat, acc = _tc_indices(zc, codebook)
    zq_cm = _sc_gather(idx_flat, codebook.T, b, c, n)

    zq = zq_cm.reshape(b, c, f, h, w)
    commit_loss = acc[0, 0] * (_COMMIT_W / (b * n * c))
    min_encoding_indices = idx_flat.reshape(-1, 1)
    return (zq, commit_loss, min_encoding_indices)


# speedup vs baseline: 1.0020x; 1.0020x over previous
import functools

import jax
import jax.numpy as jnp
from jax import lax
from jax.experimental import pallas as pl
from jax.experimental.pallas import tpu as pltpu
from jax.experimental.pallas import tpu_sc as plsc

_COMMIT_W = 0.25


def _vq_tc_body(z_ref, cb_ref, idx_ref, acc_ref):
    first = jnp.logical_and(pl.program_id(0) == 0, pl.program_id(1) == 0)

    @pl.when(first)
    def _():
        acc_ref[...] = jnp.zeros((1, 1), jnp.float32)

    zt = z_ref[0]
    cb = cb_ref[...]
    k = cb.shape[0]

    scores2 = lax.dot_general(cb, -2.0 * zt, (((1,), (0,)), ((), ())),
                              preferred_element_type=jnp.float32)
    cbsq = jnp.sum(cb * cb, axis=1, keepdims=True)
    dist = scores2 + cbsq

    m = jnp.min(dist, axis=0, keepdims=True)
    ind = (dist == m).astype(jnp.float32)

    kcol = lax.broadcasted_iota(jnp.int32, (k, 1), 0)
    khi = (kcol // 32).astype(jnp.float32)
    klo = (kcol % 32).astype(jnp.float32)
    kaug = jnp.concatenate([khi, klo], axis=1)
    qa = lax.dot_general(kaug, ind, (((0,), (0,)), ((), ())),
                         preferred_element_type=jnp.float32)
    idx = (qa[0:1, :] * 32.0 + qa[1:2, :]).astype(jnp.int32)

    acc_ref[...] += (jnp.sum(m) + jnp.sum(zt * zt)).reshape(1, 1)
    idx_ref[0] = idx


def _tc_indices(zc, codebook):
    b, c, n = zc.shape
    tile = 4096
    n_t = n // tile
    k = codebook.shape[0]
    idx_arr, acc = pl.pallas_call(
        _vq_tc_body,
        grid=(b, n_t),
        in_specs=[
            pl.BlockSpec((1, c, tile), lambda i, j: (i, 0, j)),
            pl.BlockSpec((k, c), lambda i, j: (0, 0)),
        ],
        out_specs=[
            pl.BlockSpec((1, 1, tile), lambda i, j: (i * (n // tile) + j, 0, 0)),
            pl.BlockSpec((1, 1), lambda i, j: (0, 0)),
        ],
        out_shape=[
            jax.ShapeDtypeStruct((b * n_t, 1, tile), jnp.int32),
            jax.ShapeDtypeStruct((1, 1), jnp.float32),
        ],
    )(zc, codebook)
    return idx_arr.reshape(b * n), acc


def _sc_gather_body(idx_hbm, cbt_hbm, out_hbm, idx_v, cbt_v, stage_v):
    wid = lax.axis_index("s") * 2 + lax.axis_index("c")
    ntok = 2048
    base = wid * ntok
    bi = base // 16384
    off = base % 16384
    pltpu.sync_copy(idx_hbm.at[pl.ds(base, ntok)], idx_v)
    pltpu.sync_copy(cbt_hbm, cbt_v)

    chunk = 512

    def chunk_body(ch, carry):
        def grp(g, carry2):
            iv = idx_v[pl.ds(ch * chunk + g * 16, 16)]
            for cc in range(32):
                cvec = jnp.full((16,), cc, jnp.int32)
                stage_v[cc, pl.ds(g * 16, 16)] = plsc.load_gather(
                    cbt_v, [cvec, iv])
            return carry2
        lax.fori_loop(0, chunk // 16, grp, 0, unroll=4)
        dst = pl.multiple_of(off + ch * chunk, chunk)
        pltpu.sync_copy(stage_v, out_hbm.at[bi, :, pl.ds(dst, chunk)])
        return carry
    lax.fori_loop(0, ntok // chunk, chunk_body, 0)


def _sc_gather(idx_flat, codebook_t, b, c, n):
    mesh = plsc.VectorSubcoreMesh(core_axis_name="c", subcore_axis_name="s")
    f = functools.partial(
        pl.kernel,
        out_type=jax.ShapeDtypeStruct((b, c, n), jnp.float32),
        compiler_params=pltpu.CompilerParams(needs_layout_passes=False),
        mesh=mesh,
        scratch_types=[
            pltpu.VMEM((2048,), jnp.int32),
            pltpu.VMEM((c, codebook_t.shape[1]), jnp.float32),
            pltpu.VMEM((c, 512), jnp.float32),
        ],
    )(_sc_gather_body)
    return f(idx_flat, codebook_t)


@jax.jit
def kernel(z, codebook):
    b, c, f, h, w = z.shape
    n = f * h * w
    zc = z.reshape(b, c, n)

    idx_flat, acc = _tc_indices(zc, codebook)
    zq_cm = _sc_gather(idx_flat, codebook.T, b, c, n)

    zq = zq_cm.reshape(b, c, f, h, w)
    commit_loss = acc[0, 0] * (_COMMIT_W / (b * n * c))
    min_encoding_indices = idx_flat.reshape(-1, 1)
    return (zq, commit_loss, min_encoding_indices)


# speedup vs baseline: 1.0540x; 1.0519x over previous
import functools

import jax
import jax.numpy as jnp
from jax import lax
from jax.experimental import pallas as pl
from jax.experimental.pallas import tpu as pltpu
from jax.experimental.pallas import tpu_sc as plsc

_COMMIT_W = 0.25


def _vq_tc_body(z_ref, cb_ref, idx_ref, acc_ref):
    first = jnp.logical_and(pl.program_id(0) == 0, pl.program_id(1) == 0)

    @pl.when(first)
    def _():
        acc_ref[...] = jnp.zeros((1, 1), jnp.float32)

    zt = z_ref[0]
    cb = cb_ref[...]
    k = cb.shape[0]

    scores2 = lax.dot_general(cb, -2.0 * zt, (((1,), (0,)), ((), ())),
                              preferred_element_type=jnp.float32)
    cbsq = jnp.sum(cb * cb, axis=1, keepdims=True)
    dist = scores2 + cbsq

    m = jnp.min(dist, axis=0, keepdims=True)
    idx = jnp.argmin(dist, axis=0).astype(jnp.int32)

    acc_ref[...] += (jnp.sum(m) + jnp.sum(zt * zt)).reshape(1, 1)
    idx_ref[0, 0] = idx


def _tc_indices(zc, codebook):
    b, c, n = zc.shape
    tile = 4096
    n_t = n // tile
    k = codebook.shape[0]
    idx_arr, acc = pl.pallas_call(
        _vq_tc_body,
        grid=(b, n_t),
        in_specs=[
            pl.BlockSpec((1, c, tile), lambda i, j: (i, 0, j)),
            pl.BlockSpec((k, c), lambda i, j: (0, 0)),
        ],
        out_specs=[
            pl.BlockSpec((1, 1, tile), lambda i, j: (i * (n // tile) + j, 0, 0)),
            pl.BlockSpec((1, 1), lambda i, j: (0, 0)),
        ],
        out_shape=[
            jax.ShapeDtypeStruct((b * n_t, 1, tile), jnp.int32),
            jax.ShapeDtypeStruct((1, 1), jnp.float32),
        ],
    )(zc, codebook)
    return idx_arr.reshape(b * n), acc


def _sc_gather_body(idx_hbm, cbt_hbm, out_hbm, idx_v, cbt_v, stage_v):
    wid = lax.axis_index("s") * 2 + lax.axis_index("c")
    ntok = 2048
    base = wid * ntok
    bi = base // 16384
    off = base % 16384
    pltpu.sync_copy(idx_hbm.at[pl.ds(base, ntok)], idx_v)
    pltpu.sync_copy(cbt_hbm, cbt_v)

    chunk = 512

    def chunk_body(ch, carry):
        def grp(g, carry2):
            iv = idx_v[pl.ds(ch * chunk + g * 16, 16)]
            for cc in range(32):
                cvec = jnp.full((16,), cc, jnp.int32)
                stage_v[cc, pl.ds(g * 16, 16)] = plsc.load_gather(
                    cbt_v, [cvec, iv])
            return carry2
        lax.fori_loop(0, chunk // 16, grp, 0, unroll=4)
        dst = pl.multiple_of(off + ch * chunk, chunk)
        pltpu.sync_copy(stage_v, out_hbm.at[bi, :, pl.ds(dst, chunk)])
        return carry
    lax.fori_loop(0, ntok // chunk, chunk_body, 0)


def _sc_gather(idx_flat, codebook_t, b, c, n):
    mesh = plsc.VectorSubcoreMesh(core_axis_name="c", subcore_axis_name="s")
    f = functools.partial(
        pl.kernel,
        out_type=jax.ShapeDtypeStruct((b, c, n), jnp.float32),
        compiler_params=pltpu.CompilerParams(needs_layout_passes=False),
        mesh=mesh,
        scratch_types=[
            pltpu.VMEM((2048,), jnp.int32),
            pltpu.VMEM((c, codebook_t.shape[1]), jnp.float32),
            pltpu.VMEM((c, 512), jnp.float32),
        ],
    )(_sc_gather_body)
    return f(idx_flat, codebook_t)


@jax.jit
def kernel(z, codebook):
    b, c, f, h, w = z.shape
    n = f * h * w
    zc = z.reshape(b, c, n)

    idx_flat, acc = _tc_indices(zc, codebook)
    zq_cm = _sc_gather(idx_flat, codebook.T, b, c, n)

    zq = zq_cm.reshape(b, c, f, h, w)
    commit_loss = acc[0, 0] * (_COMMIT_W / (b * n * c))
    min_encoding_indices = idx_flat.reshape(-1, 1)
    return (zq, commit_loss, min_encoding_indices)


# speedup vs baseline: 1.0588x; 1.0046x over previous
import functools

import jax
import jax.numpy as jnp
from jax import lax
from jax.experimental import pallas as pl
from jax.experimental.pallas import tpu as pltpu
from jax.experimental.pallas import tpu_sc as plsc

_COMMIT_W = 0.25


def _vq_tc_body(z_ref, cb_ref, idx_ref, acc_ref):
    first = jnp.logical_and(pl.program_id(0) == 0, pl.program_id(1) == 0)

    @pl.when(first)
    def _():
        acc_ref[...] = jnp.zeros((1, 1), jnp.float32)

    zt = z_ref[0]
    cb = cb_ref[...]

    scores2 = lax.dot_general(cb, -2.0 * zt, (((1,), (0,)), ((), ())),
                              preferred_element_type=jnp.float32)
    cbsq = jnp.sum(cb * cb, axis=1, keepdims=True)
    dist = scores2 + cbsq

    m = jnp.min(dist, axis=0, keepdims=True)
    idx = jnp.argmin(dist, axis=0).astype(jnp.int32)

    acc_ref[...] += (jnp.sum(m) + jnp.sum(zt * zt)).reshape(1, 1)
    idx_ref[0, 0] = idx


def _tc_indices(zc, codebook, b_start, b_cnt):
    _, c, n = zc.shape
    tile = 4096
    n_t = n // tile
    k = codebook.shape[0]
    idx_arr, acc = pl.pallas_call(
        _vq_tc_body,
        grid=(b_cnt, n_t),
        in_specs=[
            pl.BlockSpec((1, c, tile), lambda i, j: (i + b_start, 0, j)),
            pl.BlockSpec((k, c), lambda i, j: (0, 0)),
        ],
        out_specs=[
            pl.BlockSpec((1, 1, tile), lambda i, j: (i * (n // tile) + j, 0, 0)),
            pl.BlockSpec((1, 1), lambda i, j: (0, 0)),
        ],
        out_shape=[
            jax.ShapeDtypeStruct((b_cnt * n_t, 1, tile), jnp.int32),
            jax.ShapeDtypeStruct((1, 1), jnp.float32),
        ],
    )(zc, codebook)
    return idx_arr.reshape(b_cnt * n), acc


def _sc_gather_body(ntok, idx_hbm, cbt_hbm, out_hbm, idx_v, cbt_v, stage_v):
    wid = lax.axis_index("s") * 2 + lax.axis_index("c")
    base = wid * ntok
    bi = base // 16384
    off = base % 16384
    pltpu.sync_copy(idx_hbm.at[pl.ds(base, ntok)], idx_v)
    pltpu.sync_copy(cbt_hbm, cbt_v)

    chunk = 512

    def chunk_body(ch, carry):
        def grp(g, carry2):
            iv = idx_v[pl.ds(ch * chunk + g * 16, 16)]
            for cc in range(32):
                cvec = jnp.full((16,), cc, jnp.int32)
                stage_v[cc, pl.ds(g * 16, 16)] = plsc.load_gather(
                    cbt_v, [cvec, iv])
            return carry2
        lax.fori_loop(0, chunk // 16, grp, 0, unroll=4)
        dst = pl.multiple_of(off + ch * chunk, chunk)
        pltpu.sync_copy(stage_v, out_hbm.at[bi, :, pl.ds(dst, chunk)])
        return carry
    lax.fori_loop(0, ntok // chunk, chunk_body, 0)


def _sc_gather(idx_flat, codebook_t, b_cnt, c, n):
    ntok = (b_cnt * n) // 32
    mesh = plsc.VectorSubcoreMesh(core_axis_name="c", subcore_axis_name="s")
    f = functools.partial(
        pl.kernel,
        out_type=jax.ShapeDtypeStruct((b_cnt, c, n), jnp.float32),
        compiler_params=pltpu.CompilerParams(needs_layout_passes=False),
        mesh=mesh,
        scratch_types=[
            pltpu.VMEM((ntok,), jnp.int32),
            pltpu.VMEM((c, codebook_t.shape[1]), jnp.float32),
            pltpu.VMEM((c, 512), jnp.float32),
        ],
    )(functools.partial(_sc_gather_body, ntok))
    return f(idx_flat, codebook_t)


@jax.jit
def kernel(z, codebook):
    b, c, f, h, w = z.shape
    n = f * h * w
    zc = z.reshape(b, c, n)
    cbt = codebook.T
    half = b // 2

    idx0, acc0 = _tc_indices(zc, codebook, 0, half)
    zq0 = _sc_gather(idx0, cbt, half, c, n)
    idx1, acc1 = _tc_indices(zc, codebook, half, half)
    zq1 = _sc_gather(idx1, cbt, half, c, n)

    zq = jnp.concatenate([zq0, zq1], axis=0).reshape(b, c, f, h, w)
    commit_loss = (acc0[0, 0] + acc1[0, 0]) * (_COMMIT_W / (b * n * c))
    min_encoding_indices = jnp.concatenate([idx0, idx1]).reshape(-1, 1)
    return (zq, commit_loss, min_encoding_indices)


# speedup vs baseline: 1.1361x; 1.0730x over previous
import functools

import jax
import jax.numpy as jnp
from jax import lax
from jax.experimental import pallas as pl
from jax.experimental.pallas import tpu as pltpu
from jax.experimental.pallas import tpu_sc as plsc

_COMMIT_W = 0.25


def _vq_tc_body(z_ref, cb_ref, idx_ref, acc_ref):
    first = jnp.logical_and(pl.program_id(0) == 0, pl.program_id(1) == 0)

    @pl.when(first)
    def _():
        acc_ref[...] = jnp.zeros((1, 1), jnp.float32)

    zt = z_ref[0]
    cb = cb_ref[...]

    cbsq = jnp.sum(cb * cb, axis=1, keepdims=True)
    cba = jnp.concatenate([cb, cbsq], axis=1)
    one = jnp.ones((1, zt.shape[1]), jnp.float32)
    zta = jnp.concatenate([-2.0 * zt, one], axis=0)
    dist = lax.dot_general(cba, zta, (((1,), (0,)), ((), ())),
                           preferred_element_type=jnp.float32)

    m = jnp.min(dist, axis=0, keepdims=True)
    idx = jnp.argmin(dist, axis=0).astype(jnp.int32)

    acc_ref[...] += (jnp.sum(m) + jnp.sum(zt * zt)).reshape(1, 1)
    idx_ref[0, 0] = idx


def _tc_indices(zc, codebook, b_start, b_cnt):
    _, c, n = zc.shape
    tile = 4096
    n_t = n // tile
    k = codebook.shape[0]
    idx_arr, acc = pl.pallas_call(
        _vq_tc_body,
        grid=(b_cnt, n_t),
        in_specs=[
            pl.BlockSpec((1, c, tile), lambda i, j: (i + b_start, 0, j)),
            pl.BlockSpec((k, c), lambda i, j: (0, 0)),
        ],
        out_specs=[
            pl.BlockSpec((1, 1, tile), lambda i, j: (i * (n // tile) + j, 0, 0)),
            pl.BlockSpec((1, 1), lambda i, j: (0, 0)),
        ],
        out_shape=[
            jax.ShapeDtypeStruct((b_cnt * n_t, 1, tile), jnp.int32),
            jax.ShapeDtypeStruct((1, 1), jnp.float32),
        ],
    )(zc, codebook)
    return idx_arr.reshape(b_cnt * n), acc


def _sc_gather_body(ntok, idx_hbm, cbt_hbm, out_hbm, idx_v, cbt_v, stage_v):
    wid = lax.axis_index("s") * 2 + lax.axis_index("c")
    base = wid * ntok
    bi = base // 16384
    off = base % 16384
    pltpu.sync_copy(idx_hbm.at[pl.ds(base, ntok)], idx_v)
    pltpu.sync_copy(cbt_hbm, cbt_v)

    chunk = 512

    def chunk_body(ch, carry):
        def grp(g, carry2):
            iv = idx_v[pl.ds(ch * chunk + g * 16, 16)]
            for cc in range(32):
                cvec = jnp.full((16,), cc, jnp.int32)
                stage_v[cc, pl.ds(g * 16, 16)] = plsc.load_gather(
                    cbt_v, [cvec, iv])
            return carry2
        lax.fori_loop(0, chunk // 16, grp, 0, unroll=4)
        dst = pl.multiple_of(off + ch * chunk, chunk)
        pltpu.sync_copy(stage_v, out_hbm.at[bi, :, pl.ds(dst, chunk)])
        return carry
    lax.fori_loop(0, ntok // chunk, chunk_body, 0)


def _sc_gather(idx_flat, codebook_t, b_cnt, c, n):
    ntok = (b_cnt * n) // 32
    mesh = plsc.VectorSubcoreMesh(core_axis_name="c", subcore_axis_name="s")
    f = functools.partial(
        pl.kernel,
        out_type=jax.ShapeDtypeStruct((b_cnt, c, n), jnp.float32),
        compiler_params=pltpu.CompilerParams(needs_layout_passes=False),
        mesh=mesh,
        scratch_types=[
            pltpu.VMEM((ntok,), jnp.int32),
            pltpu.VMEM((c, codebook_t.shape[1]), jnp.float32),
            pltpu.VMEM((c, 512), jnp.float32),
        ],
    )(functools.partial(_sc_gather_body, ntok))
    return f(idx_flat, codebook_t)


@jax.jit
def kernel(z, codebook):
    b, c, f, h, w = z.shape
    n = f * h * w
    zc = z.reshape(b, c, n)
    cbt = codebook.T
    half = b // 2

    idx0, acc0 = _tc_indices(zc, codebook, 0, half)
    zq0 = _sc_gather(idx0, cbt, half, c, n)
    idx1, acc1 = _tc_indices(zc, codebook, half, half)
    zq1 = _sc_gather(idx1, cbt, half, c, n)

    zq = jnp.concatenate([zq0, zq1], axis=0).reshape(b, c, f, h, w)
    commit_loss = (acc0[0, 0] + acc1[0, 0]) * (_COMMIT_W / (b * n * c))
    min_encoding_indices = jnp.concatenate([idx0, idx1]).reshape(-1, 1)
    return (zq, commit_loss, min_encoding_indices)
